# baseline (speedup 1.0000x reference)
import jax
import jax.numpy as jnp
from jax.experimental import pallas as pl
from jax.experimental.pallas import tpu as pltpu

K = 8192
D = 49
N = 4


def _vq_body(x_ref, cb_ref, out_ref):
    xs = x_ref[...]
    cb = cb_ref[...]
    b2 = jnp.sum(cb * cb, axis=1)
    dots = jax.lax.dot_general(
        xs, cb, (((1,), (1,)), ((), ())),
        preferred_element_type=jnp.float32)
    dist = b2[None, :] - 2.0 * dots
    idx = jnp.argmin(dist, axis=1)
    onehot = (jax.lax.broadcasted_iota(jnp.int32, (N, K), 1)
              == idx[:, None]).astype(jnp.float32)
    zq = jax.lax.dot_general(
        onehot, cb, (((1,), (0,)), ((), ())),
        preferred_element_type=jnp.float32)
    out_ref[...] = xs + (zq - xs)


def kernel(x, codebook):
    out = pl.pallas_call(
        _vq_body,
        out_shape=jax.ShapeDtypeStruct((N, D), jnp.float32),
    )(x, codebook)
    return jnp.reshape(out, (4, 7, 7))
